# baseline (speedup 1.0000x reference)
import jax
import jax.numpy as jnp
from jax.experimental import pallas as pl
from jax.experimental.pallas import tpu as pltpu

_NC = 50000
_NV = 100000
_E = 800000
_HID = 64
_NLAYERS = 5
_BN_EPS = 1e-5
_EBLK = 8000
_RBLK = 6000
_VBLK = 5000


def _scatter_add_body(dst_ref, src_ref, h_ref, out_ref):
    @pl.when(pl.program_id(0) == 0)
    def _init():
        out_ref[...] = jnp.zeros_like(out_ref)

    lane_hi = jax.lax.broadcasted_iota(jnp.int32, (1, 128), 1) >= _HID

    def step(e, carry):
        d = dst_ref[0, 0, e]
        s = src_ref[0, 0, e]
        p = h_ref[pl.ds(s >> 1, 1), :]
        sbit = (s & 1) == 1
        pm = p * jnp.where(lane_hi == sbit, 1.0, 0.0)
        v = jnp.where((s & 1) == (d & 1), pm, jnp.roll(pm, _HID, axis=1))
        out_ref[pl.ds(d >> 1, 1), :] += v
        return carry

    jax.lax.fori_loop(0, _EBLK, step, 0)


def _scatter_add(dst_idx, src_idx, table_packed, n_out):
    grid = _E // _EBLK
    dst2 = dst_idx.reshape(grid, 1, _EBLK)
    src2 = src_idx.reshape(grid, 1, _EBLK)
    out = pl.pallas_call(
        _scatter_add_body,
        grid=(grid,),
        in_specs=[
            pl.BlockSpec((1, 1, _EBLK), lambda i: (i, 0, 0), memory_space=pltpu.SMEM),
            pl.BlockSpec((1, 1, _EBLK), lambda i: (i, 0, 0), memory_space=pltpu.SMEM),
            pl.BlockSpec(table_packed.shape, lambda i: (0, 0)),
        ],
        out_specs=pl.BlockSpec((n_out // 2, 2 * _HID), lambda i: (0, 0)),
        out_shape=jax.ShapeDtypeStruct((n_out // 2, 2 * _HID), jnp.float32),
    )(dst2, src2, table_packed)
    return out.reshape(n_out, _HID)


def _accum_stats(y, st_ref):
    @pl.when(pl.program_id(0) == 0)
    def _init():
        st_ref[...] = jnp.zeros_like(st_ref)

    st_ref[0:1, :] += jnp.sum(y, axis=0, keepdims=True)
    st_ref[1:2, :] += jnp.sum(y * y, axis=0, keepdims=True)


def _p1_body(eps_ref, x_ref, hp_ref, w_ref, b_ref, h1_ref, st_ref):
    xin = x_ref[...] + eps_ref[0] * hp_ref[...]
    y = jnp.dot(xin, w_ref[...], preferred_element_type=jnp.float32)
    y = y + b_ref[0:1, :]
    h1_ref[...] = y
    _accum_stats(y, st_ref)


def _p1(pooled, hprev, eps1, w1, b1):
    grid = (_NC + _NV) // _RBLK
    return pl.pallas_call(
        _p1_body,
        grid=(grid,),
        in_specs=[
            pl.BlockSpec((1,), lambda i: (0,), memory_space=pltpu.SMEM),
            pl.BlockSpec((_RBLK, _HID), lambda i: (i, 0)),
            pl.BlockSpec((_RBLK, _HID), lambda i: (i, 0)),
            pl.BlockSpec((_HID, _HID), lambda i: (0, 0)),
            pl.BlockSpec((8, _HID), lambda i: (0, 0)),
        ],
        out_specs=(
            pl.BlockSpec((_RBLK, _HID), lambda i: (i, 0)),
            pl.BlockSpec((8, _HID), lambda i: (0, 0)),
        ),
        out_shape=(
            jax.ShapeDtypeStruct((_NC + _NV, _HID), jnp.float32),
            jax.ShapeDtypeStruct((8, _HID), jnp.float32),
        ),
    )(eps1, pooled, hprev, w1, b1)


def _p2_body(x_ref, a_ref, bb_ref, w_ref, b_ref, h2_ref, st_ref):
    h = jax.nn.relu(x_ref[...] * a_ref[0:1, :] + bb_ref[0:1, :])
    y = jnp.dot(h, w_ref[...], preferred_element_type=jnp.float32)
    y = y + b_ref[0:1, :]
    h2_ref[...] = y
    _accum_stats(y, st_ref)


def _p2(h1, a1, bb1, w2, b2):
    grid = (_NC + _NV) // _RBLK
    return pl.pallas_call(
        _p2_body,
        grid=(grid,),
        in_specs=[
            pl.BlockSpec((_RBLK, _HID), lambda i: (i, 0)),
            pl.BlockSpec((8, _HID), lambda i: (0, 0)),
            pl.BlockSpec((8, _HID), lambda i: (0, 0)),
            pl.BlockSpec((_HID, _HID), lambda i: (0, 0)),
            pl.BlockSpec((8, _HID), lambda i: (0, 0)),
        ],
        out_specs=(
            pl.BlockSpec((_RBLK, _HID), lambda i: (i, 0)),
            pl.BlockSpec((8, _HID), lambda i: (0, 0)),
        ),
        out_shape=(
            jax.ShapeDtypeStruct((_NC + _NV, _HID), jnp.float32),
            jax.ShapeDtypeStruct((8, _HID), jnp.float32),
        ),
    )(h1, a1, bb1, w2, b2)


def _p3_body(x_ref, a_ref, bb_ref, h_ref):
    h_ref[...] = jax.nn.relu(x_ref[...] * a_ref[0:1, :] + bb_ref[0:1, :])


def _p3(h2, a2, bb2):
    grid = (_NC + _NV) // _RBLK
    return pl.pallas_call(
        _p3_body,
        grid=(grid,),
        in_specs=[
            pl.BlockSpec((_RBLK, _HID), lambda i: (i, 0)),
            pl.BlockSpec((8, _HID), lambda i: (0, 0)),
            pl.BlockSpec((8, _HID), lambda i: (0, 0)),
        ],
        out_specs=pl.BlockSpec((_RBLK, _HID), lambda i: (i, 0)),
        out_shape=jax.ShapeDtypeStruct((_NC + _NV, _HID), jnp.float32),
    )(h2, a2, bb2)


def _final_body(x_ref, w_ref, b_ref, o_ref):
    logits = jnp.dot(x_ref[...], w_ref[...], preferred_element_type=jnp.float32)
    logits = logits + b_ref[0:1, :]
    m = jnp.max(logits, axis=1, keepdims=True)
    e = jnp.exp(logits - m)
    o_ref[...] = e / jnp.sum(e, axis=1, keepdims=True)


def _final(hv, wp, bp):
    grid = _NV // _VBLK
    return pl.pallas_call(
        _final_body,
        grid=(grid,),
        in_specs=[
            pl.BlockSpec((_VBLK, _HID), lambda i: (i, 0)),
            pl.BlockSpec((_HID, 8), lambda i: (0, 0)),
            pl.BlockSpec((8, 8), lambda i: (0, 0)),
        ],
        out_specs=pl.BlockSpec((_VBLK, 8), lambda i: (i, 0)),
        out_shape=jax.ShapeDtypeStruct((_NV, 8), jnp.float32),
    )(hv, wp, bp)


def _rep8(v):
    return jnp.broadcast_to(v.astype(jnp.float32)[None, :], (8, _HID))


def _bn_coeffs(st, n, g, b):
    mu = st[0] / n
    var = st[1] / n - mu * mu
    a = g / jnp.sqrt(var + _BN_EPS)
    return _rep8(a), _rep8(b - mu * a)


def kernel(edge_clause, edge_var, params):
    zeros_idx = jnp.zeros((_E,), jnp.int32)
    e0_table = jnp.zeros((8, 2 * _HID), jnp.float32).at[0, 0].set(1.0)

    hc = _scatter_add(edge_clause, zeros_idx, e0_table, _NC)
    hv = _scatter_add(edge_var, zeros_idx, e0_table, _NV)
    h_cat = jnp.concatenate([hc, hv], axis=0)

    n = jnp.float32(_NC + _NV)
    for l in range(_NLAYERS - 1):
        p = params['layers'][l]
        h_clause = h_cat[:_NC].reshape(_NC // 2, 2 * _HID)
        h_var = h_cat[_NC:].reshape(_NV // 2, 2 * _HID)
        cp = _scatter_add(edge_clause, edge_var, h_var, _NC)
        vp = _scatter_add(edge_var, edge_clause, h_clause, _NV)
        pooled = jnp.concatenate([cp, vp], axis=0)

        w1 = p['W1']
        if w1.shape[0] < _HID:
            w1 = jnp.zeros((_HID, _HID), jnp.float32).at[: w1.shape[0]].set(w1)
        eps1 = (1.0 + params['eps'][l]).reshape(1)

        h1, st1 = _p1(pooled, h_cat, eps1, w1, _rep8(p['b1']))
        a1, bb1 = _bn_coeffs(st1, n, p['g1'], p['bt1'])
        h2, st2 = _p2(h1, a1, bb1, p['W2'], _rep8(p['b2']))
        a2, bb2 = _bn_coeffs(st2, n, p['gbn'], p['bbn'])
        h_cat = _p3(h2, a2, bb2)

    wp = jnp.full((_HID, 8), 0.0, jnp.float32).at[:, :2].set(params['fc1W'])
    bp = jnp.full((8, 8), -1e30, jnp.float32).at[:, :2].set(
        jnp.broadcast_to(params['fc1b'][None, :], (8, 2)))
    probs = _final(h_cat[_NC:], wp, bp)
    return probs[:, :2]
